# baseline (speedup 1.0000x reference)
import jax
import jax.numpy as jnp
from jax.experimental import pallas as pl
from jax.experimental.pallas import tpu as pltpu
from jax.experimental.pallas import tpu_sc as plsc

_SHIFTS = (0, 1, 3)
_RB = 80
_STRIPE = 512
_V = 16


def _sc_kernel(x_hbm, o_hbm, buf, stage, sems):
    n, b = x_hbm.shape
    core = jax.lax.axis_index("core")
    sub = jax.lax.axis_index("subcore")
    c0 = (core * 16 + sub) * _STRIPE
    cols = pl.ds(c0, _STRIPE)
    n_blk = n // _RB

    for i in range(n_blk):
        a = i * _RB
        cp_main = pltpu.make_async_copy(
            x_hbm.at[pl.ds(a, _RB), cols], buf.at[pl.ds(0, _RB)], sems.at[0])
        cp_head = pltpu.make_async_copy(
            x_hbm.at[pl.ds((a + _RB) % n, 8), cols], buf.at[pl.ds(_RB, 8)],
            sems.at[1])
        cp_main.start()
        cp_head.start()
        cp_main.wait()
        cp_head.wait()

        cp0 = pltpu.make_async_copy(
            buf.at[pl.ds(0, _RB)], o_hbm.at[0, pl.ds(a, _RB), cols],
            sems.at[2])
        cp0.start()

        prev = None
        for k, s in enumerate(_SHIFTS):
            if s == 0:
                continue
            if prev is not None:
                prev.wait()

            @pl.loop(0, _RB)
            def _(r, s=s):
                @pl.loop(0, _STRIPE, step=_V)
                def _(c, r=r, s=s):
                    stage[r, pl.ds(c, _V)] = buf[r + s, pl.ds(c, _V)]

            cpk = pltpu.make_async_copy(
                stage.at[pl.ds(0, _RB)], o_hbm.at[k, pl.ds(a, _RB), cols],
                sems.at[3])
            cpk.start()
            prev = cpk
        cp0.wait()
        prev.wait()


def kernel(x_all):
    b, n = x_all.shape
    g = len(_SHIFTS)
    xt = x_all.T
    mesh = plsc.VectorSubcoreMesh(core_axis_name="core",
                                  subcore_axis_name="subcore")
    yt = pl.kernel(
        _sc_kernel,
        out_type=jax.ShapeDtypeStruct((g, n, b), x_all.dtype),
        mesh=mesh,
        scratch_types=[
            pltpu.VMEM((_RB + 8, _STRIPE), jnp.float32),
            pltpu.VMEM((_RB, _STRIPE), jnp.float32),
            pltpu.SemaphoreType.DMA((4,)),
        ],
    )(xt)
    return yt.transpose(2, 1, 0)


# speedup vs baseline: 1.0873x; 1.0873x over previous
import jax
import jax.numpy as jnp
from jax.experimental import pallas as pl
from jax.experimental.pallas import tpu as pltpu
from jax.experimental.pallas import tpu_sc as plsc

_SHIFTS = (0, 1, 3)
_RB = 80
_HALF = 256
_V = 16


def _sc_kernel(x_hbm, o_hbm, buf_a, buf_b, st1a, st3a, st1b, st3b, sems):
    n, b = x_hbm.shape
    core = jax.lax.axis_index("core")
    sub = jax.lax.axis_index("subcore")
    base = (core * 16 + sub) * (2 * _HALF)
    n_pair = n // (2 * _RB)

    def do_block(a, cols, buf, st1, st3, sem0):
        cp_main = pltpu.make_async_copy(
            x_hbm.at[pl.ds(a, _RB), cols], buf.at[pl.ds(0, _RB)], sems.at[0])
        cp_head = pltpu.make_async_copy(
            x_hbm.at[pl.ds((a + _RB) % n, 8), cols], buf.at[pl.ds(_RB, 8)],
            sems.at[1])
        cp_main.start()
        cp_head.start()
        cp_main.wait()
        cp_head.wait()
        cps = [pltpu.make_async_copy(
            buf.at[pl.ds(0, _RB)], o_hbm.at[0, pl.ds(a, _RB), cols],
            sems.at[sem0])]
        cps[0].start()
        for j, (k, s, stage) in enumerate(((1, 1, st1), (2, 3, st3))):
            @pl.loop(0, _RB)
            def _(r, s=s, stage=stage):
                for c in range(0, _HALF, _V):
                    stage[r, pl.ds(c, _V)] = buf[r + s, pl.ds(c, _V)]

            cpk = pltpu.make_async_copy(
                stage.at[pl.ds(0, _RB)], o_hbm.at[k, pl.ds(a, _RB), cols],
                sems.at[sem0 + 1 + j])
            cpk.start()
            cps.append(cpk)
        return cps

    for half in range(2):
        cols = pl.ds(base + half * _HALF, _HALF)

        @pl.loop(0, n_pair)
        def _(t, cols=cols):
            a = pl.multiple_of(t * (2 * _RB), 8)
            cps_a = do_block(a, cols, buf_a, st1a, st3a, 2)
            cps_b = do_block(a + _RB, cols, buf_b, st1b, st3b, 5)
            for cp in cps_a + cps_b:
                cp.wait()


def kernel(x_all):
    b, n = x_all.shape
    g = len(_SHIFTS)
    xt = x_all.T
    mesh = plsc.VectorSubcoreMesh(core_axis_name="core",
                                  subcore_axis_name="subcore")
    yt = pl.kernel(
        _sc_kernel,
        out_type=jax.ShapeDtypeStruct((g, n, b), x_all.dtype),
        mesh=mesh,
        scratch_types=[
            pltpu.VMEM((_RB + 8, _HALF), jnp.float32),
            pltpu.VMEM((_RB + 8, _HALF), jnp.float32),
            pltpu.VMEM((_RB, _HALF), jnp.float32),
            pltpu.VMEM((_RB, _HALF), jnp.float32),
            pltpu.VMEM((_RB, _HALF), jnp.float32),
            pltpu.VMEM((_RB, _HALF), jnp.float32),
            pltpu.SemaphoreType.DMA((8,)),
        ],
    )(xt)
    return yt.transpose(2, 1, 0)


# speedup vs baseline: 3.9474x; 3.6305x over previous
import jax
import jax.numpy as jnp
from jax.experimental import pallas as pl
from jax.experimental.pallas import tpu as pltpu

_SHIFTS = (0, 1, 3)
_COLS = 1024


def _tok_kernel(x_ref, o_ref):
    x = x_ref[...]
    for k, s in enumerate(_SHIFTS):
        o_ref[k] = jnp.concatenate([x[s:], x[:s]], axis=0) if s else x


def kernel(x_all):
    b, n = x_all.shape
    g = len(_SHIFTS)
    xt = x_all.T
    yt = pl.pallas_call(
        _tok_kernel,
        grid=(b // _COLS,),
        in_specs=[pl.BlockSpec((n, _COLS), lambda i: (0, i))],
        out_specs=pl.BlockSpec((g, n, _COLS), lambda i: (0, 0, i)),
        out_shape=jax.ShapeDtypeStruct((g, n, b), x_all.dtype),
        compiler_params=pltpu.CompilerParams(
            dimension_semantics=("parallel",),
        ),
    )(xt)
    return yt.transpose(2, 1, 0)
